# Initial kernel scaffold; baseline (speedup 1.0000x reference)
import jax
import jax.numpy as jnp
from jax.experimental import pallas as pl


def kernel(z, codebook):
    raise NotImplementedError("write your pallas kernel here")


# speedup vs baseline: 1.2275x; 1.2275x over previous
import functools

import jax
import jax.numpy as jnp
from jax import lax
from jax.experimental import pallas as pl
from jax.experimental.pallas import tpu as pltpu

K_BINS = 1024
EMB = 64
TN = 512


def _vq_body(z_ref, cb_ref, xq_ref, xl_ref, fit_ref, commit_ref):
    i = pl.program_id(0)
    x = z_ref[...]
    cb = cb_ref[...]
    xsq = jnp.sum(x * x, axis=1, keepdims=True)
    ksq = jnp.sum(cb * cb, axis=1)[None, :]
    xk = lax.dot_general(x, cb, (((1,), (1,)), ((), ())),
                         preferred_element_type=jnp.float32)
    dist = xsq - 2.0 * xk + ksq
    mind = jnp.min(dist, axis=1)
    iota = lax.broadcasted_iota(jnp.int32, dist.shape, 1)
    idx = jnp.min(jnp.where(dist <= mind[:, None], iota, K_BINS), axis=1)
    onehot = (iota == idx[:, None]).astype(jnp.float32)
    xd = lax.dot_general(onehot, cb, (((1,), (0,)), ((), ())),
                         preferred_element_type=jnp.float32)
    xq_ref[...] = x + (xd - x)
    xl_ref[...] = idx

    @pl.when(i == 0)
    def _init():
        fit_ref[...] = jnp.zeros((1, 1), jnp.float32)
        commit_ref[...] = jnp.zeros((1, 1), jnp.float32)

    fit_ref[...] += jnp.sum(mind).reshape(1, 1)
    commit_ref[...] += jnp.sum((xd - x) ** 2).reshape(1, 1)


@jax.jit
def kernel(z, codebook):
    B, T, D = z.shape
    N = B * T
    x = z.reshape(N, D)
    grid = (N // TN,)
    xq, xl, fit_s, commit_s = pl.pallas_call(
        _vq_body,
        grid=grid,
        in_specs=[
            pl.BlockSpec((TN, D), lambda i: (i, 0)),
            pl.BlockSpec((K_BINS, D), lambda i: (0, 0)),
        ],
        out_specs=[
            pl.BlockSpec((TN, D), lambda i: (i, 0)),
            pl.BlockSpec((TN,), lambda i: (i,)),
            pl.BlockSpec((1, 1), lambda i: (0, 0)),
            pl.BlockSpec((1, 1), lambda i: (0, 0)),
        ],
        out_shape=[
            jax.ShapeDtypeStruct((N, D), jnp.float32),
            jax.ShapeDtypeStruct((N,), jnp.int32),
            jax.ShapeDtypeStruct((1, 1), jnp.float32),
            jax.ShapeDtypeStruct((1, 1), jnp.float32),
        ],
    )(x, codebook)
    fit = fit_s[0, 0] / N
    commit_loss = commit_s[0, 0] / (N * D)
    return xq.reshape(B, T, D), commit_loss, fit, xl.reshape(B, T)
